# baseline (speedup 1.0000x reference)
import functools

import jax
import jax.numpy as jnp
from jax import lax
from jax.experimental import pallas as pl
from jax.experimental.pallas import tpu as pltpu
from jax.experimental.pallas import tpu_sc as plsc

NODES = 10000
EDGES = 320000
IN_C = 128
OUT_C = 128
NC = 2
NS = 16
NTILES = NC * NS
EPT = EDGES // NTILES
LANES = 16
VECS_E = EPT // LANES
VECS_N = NODES // LANES
WIN = 10112

_mesh = plsc.VectorSubcoreMesh(core_axis_name="c", subcore_axis_name="s")
_sc_params = pltpu.CompilerParams(needs_layout_passes=False)


def _wid():
    return lax.axis_index("s") * NC + lax.axis_index("c")


def _zero(ref):
    zeros = jnp.zeros((LANES,), jnp.float32)

    @plsc.parallel_loop(0, NODES, step=LANES, unroll=8)
    def _(s):
        ref[pl.ds(s, LANES)] = zeros


@functools.partial(
    pl.kernel,
    out_type=jax.ShapeDtypeStruct((NTILES, NODES), jnp.float32),
    mesh=_mesh,
    compiler_params=_sc_params,
    scratch_types=[
        pltpu.VMEM((2, WIN), jnp.int32),
        pltpu.VMEM((EPT,), jnp.float32),
        pltpu.VMEM((NODES,), jnp.float32),
    ],
)
def _sc_deg(ei_hbm, ew_hbm, out_hbm, rc_v, ew_v, acc_v):
    base = _wid() * EPT
    win = (base // 128) * 128
    off = base - win
    pltpu.sync_copy(ei_hbm.at[:, pl.ds(win, WIN)], rc_v)
    pltpu.sync_copy(ew_hbm.at[pl.ds(base, EPT)], ew_v)
    _zero(acc_v)

    @plsc.parallel_loop(0, EPT, step=LANES, unroll=8)
    def _(s):
        idx = rc_v[1, pl.ds(off + s, LANES)]
        w = ew_v[pl.ds(s, LANES)]
        plsc.addupdate_scatter(acc_v, [idx], w)
    pltpu.sync_copy(acc_v, out_hbm.at[_wid()])


def _tc_norm_body(degp_ref, dis_ref, inv_ref):
    deg = jnp.sum(degp_ref[...], axis=0, keepdims=True) + 1.0
    pos = deg > 0
    dis_ref[...] = jnp.where(pos, lax.rsqrt(deg), 0.0)
    inv_ref[...] = jnp.where(pos, 1.0 / deg, 0.0)


_tc_norm = pl.pallas_call(
    _tc_norm_body,
    out_shape=[
        jax.ShapeDtypeStruct((1, NODES), jnp.float32),
        jax.ShapeDtypeStruct((1, NODES), jnp.float32),
    ],
)


@functools.partial(
    pl.kernel,
    out_type=[
        jax.ShapeDtypeStruct((NTILES, NODES), jnp.float32),
        jax.ShapeDtypeStruct((NTILES, NODES), jnp.float32),
    ],
    mesh=_mesh,
    compiler_params=_sc_params,
    scratch_types=[
        pltpu.VMEM((2, WIN), jnp.int32),
        pltpu.VMEM((EPT,), jnp.float32),
        pltpu.VMEM((NODES,), jnp.float32),
        pltpu.VMEM((NODES,), jnp.float32),
        pltpu.VMEM((NODES,), jnp.float32),
    ],
)
def _sc_edges(ei_hbm, ew_hbm, dis_hbm, out_a, out_c,
              rc_v, ew_v, dis_v, acc_a, acc_c):
    base = _wid() * EPT
    win = (base // 128) * 128
    off = base - win
    pltpu.sync_copy(dis_hbm.at[0], dis_v)
    pltpu.sync_copy(ei_hbm.at[:, pl.ds(win, WIN)], rc_v)
    pltpu.sync_copy(ew_hbm.at[pl.ds(base, EPT)], ew_v)
    _zero(acc_a)
    _zero(acc_c)

    @plsc.parallel_loop(0, EPT, step=LANES, unroll=16)
    def _(s):
        r = rc_v[0, pl.ds(off + s, LANES)]
        c = rc_v[1, pl.ds(off + s, LANES)]
        w = ew_v[pl.ds(s, LANES)]
        dr = plsc.load_gather(dis_v, [r])
        dc = plsc.load_gather(dis_v, [c])
        nrm = dr * w * dc
        plsc.addupdate_scatter(acc_a, [c], nrm)
        plsc.addupdate_scatter(acc_c, [r], nrm)
    pltpu.sync_copy(acc_a, out_a.at[_wid()])
    pltpu.sync_copy(acc_c, out_c.at[_wid()])


def _tc_tail_body(ap_ref, cp_ref, inv_ref, w1_ref, b1_ref, w2_ref, b2_ref,
                  out_ref):
    inv = inv_ref[...]
    a = jnp.sum(ap_ref[...], axis=0, keepdims=True) + inv
    c = jnp.sum(cp_ref[...], axis=0, keepdims=True) + inv
    ones = jnp.ones((IN_C, 1), jnp.float32)
    s1 = lax.dot_general(w1_ref[...], ones, (((0,), (0,)), ((), ())))
    h1 = jnp.maximum(s1 * a + b1_ref[...], 0.0)
    v = jnp.sum(h1 * c, axis=1, keepdims=True)
    out = lax.dot_general(v, w2_ref[...], (((0,), (0,)), ((), ())))
    out_ref[...] = out * (1.0 / NODES) + b2_ref[...]


_tc_tail = pl.pallas_call(
    _tc_tail_body,
    out_shape=jax.ShapeDtypeStruct((1, OUT_C), jnp.float32),
)


def kernel(x, edge_index, edge_attr, W1, b1, W2, b2):
    del x
    deg_part = _sc_deg(edge_index, edge_attr)
    dis, inv = _tc_norm(deg_part)
    a_part, c_part = _sc_edges(edge_index, edge_attr, dis)
    return _tc_tail(a_part, c_part, inv, W1, jnp.reshape(b1, (IN_C, 1)),
                    W2, jnp.reshape(b2, (1, OUT_C)))


# speedup vs baseline: 1.0054x; 1.0054x over previous
import functools

import jax
import jax.numpy as jnp
from jax import lax
from jax.experimental import pallas as pl
from jax.experimental.pallas import tpu as pltpu
from jax.experimental.pallas import tpu_sc as plsc

NODES = 10000
EDGES = 320000
IN_C = 128
OUT_C = 128
NC = 2
NS = 16
NTILES = NC * NS
EPT = EDGES // NTILES
LANES = 16
VECS_E = EPT // LANES
VECS_N = NODES // LANES
WIN = 10112

_mesh = plsc.VectorSubcoreMesh(core_axis_name="c", subcore_axis_name="s")
_sc_params = pltpu.CompilerParams(needs_layout_passes=False)


def _wid():
    return lax.axis_index("s") * NC + lax.axis_index("c")


def _zero(ref):
    zeros = jnp.zeros((LANES,), jnp.float32)

    @plsc.parallel_loop(0, NODES, step=LANES, unroll=8)
    def _(s):
        ref[pl.ds(s, LANES)] = zeros


@functools.partial(
    pl.kernel,
    out_type=jax.ShapeDtypeStruct((NTILES, NODES), jnp.float32),
    mesh=_mesh,
    compiler_params=_sc_params,
    scratch_types=[
        pltpu.VMEM((2, WIN), jnp.int32),
        pltpu.VMEM((EPT,), jnp.float32),
        pltpu.VMEM((NODES,), jnp.float32),
    ],
)
def _sc_deg(ei_hbm, ew_hbm, out_hbm, rc_v, ew_v, acc_v):
    base = _wid() * EPT
    win = (base // 128) * 128
    off = base - win
    pltpu.sync_copy(ei_hbm.at[:, pl.ds(win, WIN)], rc_v)
    pltpu.sync_copy(ew_hbm.at[pl.ds(base, EPT)], ew_v)
    _zero(acc_v)

    @plsc.parallel_loop(0, EPT, step=LANES, unroll=8)
    def _(s):
        idx = rc_v[1, pl.ds(off + s, LANES)]
        w = ew_v[pl.ds(s, LANES)]
        plsc.addupdate_scatter(acc_v, [idx], w)
    pltpu.sync_copy(acc_v, out_hbm.at[_wid()])


def _tc_norm_body(degp_ref, dis_ref, inv_ref):
    deg = jnp.sum(degp_ref[...], axis=0, keepdims=True) + 1.0
    pos = deg > 0
    dis_ref[...] = jnp.reshape(jnp.where(pos, lax.rsqrt(deg), 0.0), (NODES,))
    inv_ref[...] = jnp.reshape(jnp.where(pos, 1.0 / deg, 0.0), (NODES,))


_tc_norm = pl.pallas_call(
    _tc_norm_body,
    out_shape=[
        jax.ShapeDtypeStruct((NODES,), jnp.float32),
        jax.ShapeDtypeStruct((NODES,), jnp.float32),
    ],
)


@functools.partial(
    pl.kernel,
    out_type=[
        jax.ShapeDtypeStruct((NTILES, NODES), jnp.float32),
        jax.ShapeDtypeStruct((NTILES, NODES), jnp.float32),
    ],
    mesh=_mesh,
    compiler_params=_sc_params,
    scratch_types=[
        pltpu.VMEM((2, WIN), jnp.int32),
        pltpu.VMEM((EPT,), jnp.float32),
        pltpu.VMEM((NODES,), jnp.float32),
        pltpu.VMEM((NODES,), jnp.float32),
        pltpu.VMEM((NODES,), jnp.float32),
    ],
)
def _sc_edges(ei_hbm, ew_hbm, dis_hbm, out_a, out_c,
              rc_v, ew_v, dis_v, acc_a, acc_c):
    base = _wid() * EPT
    win = (base // 128) * 128
    off = base - win
    pltpu.sync_copy(dis_hbm, dis_v)
    pltpu.sync_copy(ei_hbm.at[:, pl.ds(win, WIN)], rc_v)
    pltpu.sync_copy(ew_hbm.at[pl.ds(base, EPT)], ew_v)
    _zero(acc_a)
    _zero(acc_c)

    @plsc.parallel_loop(0, EPT, step=LANES, unroll=8)
    def _(s):
        r = rc_v[0, pl.ds(off + s, LANES)]
        c = rc_v[1, pl.ds(off + s, LANES)]
        w = ew_v[pl.ds(s, LANES)]
        dr = plsc.load_gather(dis_v, [r])
        dc = plsc.load_gather(dis_v, [c])
        nrm = dr * w * dc
        plsc.addupdate_scatter(acc_a, [c], nrm)
        plsc.addupdate_scatter(acc_c, [r], nrm)
    pltpu.sync_copy(acc_a, out_a.at[_wid()])
    pltpu.sync_copy(acc_c, out_c.at[_wid()])


def _tc_tail_body(ap_ref, cp_ref, inv_ref, w1_ref, b1_ref, w2_ref, b2_ref,
                  out_ref):
    inv = jnp.reshape(inv_ref[...], (1, NODES))
    a = jnp.sum(ap_ref[...], axis=0, keepdims=True) + inv
    c = jnp.sum(cp_ref[...], axis=0, keepdims=True) + inv
    ones = jnp.ones((IN_C, 1), jnp.float32)
    s1 = lax.dot_general(w1_ref[...], ones, (((0,), (0,)), ((), ())))
    h1 = jnp.maximum(s1 * a + b1_ref[...], 0.0)
    v = jnp.sum(h1 * c, axis=1, keepdims=True)
    out = lax.dot_general(v, w2_ref[...], (((0,), (0,)), ((), ())))
    out_ref[...] = out * (1.0 / NODES) + b2_ref[...]


_tc_tail = pl.pallas_call(
    _tc_tail_body,
    out_shape=jax.ShapeDtypeStruct((1, OUT_C), jnp.float32),
)


def kernel(x, edge_index, edge_attr, W1, b1, W2, b2):
    del x
    deg_part = _sc_deg(edge_index, edge_attr)
    dis, inv = _tc_norm(deg_part)
    a_part, c_part = _sc_edges(edge_index, edge_attr, dis)
    return _tc_tail(a_part, c_part, inv, W1, jnp.reshape(b1, (IN_C, 1)),
                    W2, jnp.reshape(b2, (1, OUT_C)))
